# baseline (speedup 1.0000x reference)
import functools

import jax
import jax.numpy as jnp
from jax import lax
from jax.experimental import pallas as pl
from jax.experimental.pallas import tpu as pltpu
from jax.experimental.pallas import tpu_sc as plsc

_N = 10000
_E = 160000
_D = 256
_H = 128
_NSUB = 16
_K = 80
_EPS = _E // _NSUB
_NCH = _EPS // _K
_ROWS = 624
_EXTRA = _N - _NSUB * _ROWS

_MESH = plsc.VectorSubcoreMesh(core_axis_name="c", subcore_axis_name="s")


def _zero_slices(zsrc_hbm, dst_spmem, rbase, s):
    pltpu.sync_copy(zsrc_hbm.at[pl.ds(0, _ROWS)],
                    dst_spmem.at[pl.ds(rbase, _ROWS)])

    @pl.when(s == _NSUB - 1)
    def _():
        pltpu.sync_copy(zsrc_hbm.at[pl.ds(0, _EXTRA)],
                        dst_spmem.at[pl.ds(_NSUB * _ROWS, _EXTRA)])


def _copy_out_slices(acc_spmem, bounce, out_ref, rbase, s):
    del bounce
    pltpu.sync_copy(acc_spmem.at[pl.ds(rbase, _ROWS)],
                    out_ref.at[pl.ds(rbase, _ROWS)])

    @pl.when(s == _NSUB - 1)
    def _():
        r0 = _NSUB * _ROWS
        pltpu.sync_copy(acc_spmem.at[pl.ds(r0, _EXTRA)],
                        out_ref.at[pl.ds(r0, _EXTRA)])


def _make_agg():
    out_type = [
        jax.ShapeDtypeStruct((_N, _H), jnp.float32),
        jax.ShapeDtypeStruct((_N, _H), jnp.float32),
    ]
    scratch_types = [
        pltpu.VMEM((2, _K), jnp.int32),
        pltpu.VMEM((2, _K), jnp.int32),
        pltpu.VMEM((2, _K), jnp.int32),
        pltpu.VMEM((2, _K), jnp.int32),
        pltpu.VMEM((_K, _H), jnp.float32),
        pltpu.VMEM((_K, _H), jnp.float32),
        pltpu.VMEM_SHARED((_N, _H), jnp.float32),
        pltpu.SemaphoreType.DMA,
        pltpu.SemaphoreType.DMA,
        pltpu.SemaphoreType.DMA,
        pltpu.SemaphoreType.DMA,
        pltpu.SemaphoreType.DMA,
        pltpu.SemaphoreType.DMA,
        pltpu.SemaphoreType.DMA,
        pltpu.SemaphoreType.DMA,
    ]

    @functools.partial(pl.kernel, mesh=_MESH, out_type=out_type,
                       scratch_types=scratch_types)
    def agg(ha, hb, edges, zrows, out_a, out_b,
            ib0, ib1, ib2, ib3, st0, st1, acc,
            semi0, semi1, semi2, semi3, semg0, semg1, sems0, sems1):
        c = lax.axis_index("c")
        s = lax.axis_index("s")
        rbase = s * _ROWS
        my_edges = edges.at[s]

        _zero_slices(zrows, acc, rbase, s)

        plsc.subcore_barrier()

        def run(h):
            ibs = (ib0, ib1, ib2, ib3)
            semis = (semi0, semi1, semi2, semi3)
            sts = (st0, st1)
            semgs = (semg0, semg1)
            semss = (sems0, sems1)

            def idx_load(i, q):
                pltpu.async_copy(my_edges.at[i], ibs[q], semis[q])

            def idx_wait(q):
                pltpu.make_async_copy(my_edges.at[0], ibs[q],
                                      semis[q]).wait()

            def gather(q, p):
                pltpu.async_copy(h.at[ibs[q].at[0]], sts[p], semgs[p])

            def gather_wait(p):
                pltpu.make_async_copy(h.at[ibs[0].at[0]], sts[p],
                                      semgs[p]).wait()

            def scat_start(q, p):
                pltpu.make_async_copy(sts[p], acc.at[ibs[q].at[1]],
                                      semss[p]).start(add=True)

            def scat_wait(p):
                pltpu.make_async_copy(sts[p], acc.at[ibs[0].at[1]],
                                      semss[p]).wait()

            idx_load(0, 0)
            idx_load(1, 1)
            idx_load(2, 2)
            idx_load(3, 3)
            idx_wait(0)
            gather(0, 0)
            idx_wait(1)
            gather(1, 1)

            def group(j, carry):
                a = 4 * j
                gather_wait(0)
                scat_start(0, 0)
                gather_wait(1)
                scat_start(1, 1)
                scat_wait(0)

                @pl.when(a + 4 < _NCH)
                def _():
                    idx_load(a + 4, 0)

                idx_wait(2)
                gather(2, 0)
                scat_wait(1)

                @pl.when(a + 5 < _NCH)
                def _():
                    idx_load(a + 5, 1)

                idx_wait(3)
                gather(3, 1)
                gather_wait(0)
                scat_start(2, 0)
                gather_wait(1)
                scat_start(3, 1)
                scat_wait(0)

                @pl.when(a + 6 < _NCH)
                def _():
                    idx_load(a + 6, 2)

                @pl.when(a + 4 < _NCH)
                def _():
                    idx_wait(0)
                    gather(0, 0)

                scat_wait(1)

                @pl.when(a + 7 < _NCH)
                def _():
                    idx_load(a + 7, 3)

                @pl.when(a + 5 < _NCH)
                def _():
                    idx_wait(1)
                    gather(1, 1)

                return carry

            lax.fori_loop(0, _NCH // 4, group, 0)
            gather_wait(0)
            scat_start(0, 0)
            scat_wait(0)

        @pl.when(c == 0)
        def _():
            run(ha)

        @pl.when(c == 1)
        def _():
            run(hb)

        plsc.subcore_barrier()

        @pl.when(c == 0)
        def _():
            _copy_out_slices(acc, st0, out_a, rbase, s)

        @pl.when(c == 1)
        def _():
            _copy_out_slices(acc, st0, out_b, rbase, s)

    return agg


def _make_deg():
    kd = 40
    nchd = _E // 2 // _NSUB // kd
    out_type = [
        jax.ShapeDtypeStruct((_N, _H), jnp.float32),
        jax.ShapeDtypeStruct((_N, _H), jnp.float32),
    ]
    scratch_types = [
        pltpu.VMEM((kd,), jnp.int32),
        pltpu.VMEM((kd,), jnp.int32),
        pltpu.VMEM((kd, _H), jnp.float32),
        pltpu.VMEM_SHARED((_N, _H), jnp.float32),
        pltpu.SemaphoreType.DMA,
        pltpu.SemaphoreType.DMA,
        pltpu.SemaphoreType.DMA,
        pltpu.SemaphoreType.DMA,
    ]

    @functools.partial(pl.kernel, mesh=_MESH, out_type=out_type,
                       scratch_types=scratch_types)
    def deg(dst3, zrows, ones_h, deg_a, deg_b, ib0, ib1, st, dacc,
            semi0, semi1, sems0, sems1):
        c = lax.axis_index("c")
        s = lax.axis_index("s")
        rbase = s * _ROWS
        w = c * _NSUB + s
        my_dst = dst3.at[w]

        _zero_slices(zrows, dacc, rbase, s)
        pltpu.sync_copy(ones_h, st)
        plsc.subcore_barrier()

        def idx_load(i, ib, sem):
            pltpu.async_copy(my_dst.at[i], ib, sem)

        def idx_wait(ib, sem):
            pltpu.make_async_copy(my_dst.at[0], ib, sem).wait()

        def scat_start(ib, sem):
            pltpu.make_async_copy(st, dacc.at[ib], sem).start(add=True)

        def scat_wait(ib, sem):
            pltpu.make_async_copy(st, dacc.at[ib], sem).wait()

        pltpu.sync_copy(my_dst.at[0], ib0)
        idx_load(1, ib1, semi1)
        scat_start(ib0, sems0)

        def step(j, carry):
            i0 = 2 * j
            idx_wait(ib1, semi1)
            scat_start(ib1, sems1)
            scat_wait(ib0, sems0)
            idx_load(i0 + 2, ib0, semi0)
            idx_wait(ib0, semi0)
            scat_start(ib0, sems0)
            scat_wait(ib1, sems1)

            @pl.when(i0 + 3 < nchd)
            def _():
                idx_load(i0 + 3, ib1, semi1)

            return carry

        lax.fori_loop(0, (nchd - 1) // 2, step, 0)
        scat_wait(ib0, sems0)

        plsc.subcore_barrier()

        @pl.when(c == 0)
        def _():
            _copy_out_slices(dacc, st, deg_a, rbase, s)

        @pl.when(c == 1)
        def _():
            _copy_out_slices(dacc, st, deg_b, rbase, s)

    return deg


_AGG = _make_agg()
_DEG = _make_deg()

_BN = 1000


def _make_update(relu, final):
    def body(aa, ab, dga, dgb, ha, hb, wna, wnb, wsa, wsb, bb, *outs):
        degc = dga[...][:, 0:1] + dgb[...][:, 0:1]
        inv = 1.0 / jnp.maximum(degc, 1.0)
        acc = jnp.dot(aa[...] * inv, wna[...],
                      preferred_element_type=jnp.float32)
        acc = acc + jnp.dot(ab[...] * inv, wnb[...],
                            preferred_element_type=jnp.float32)
        acc = acc + jnp.dot(ha[...], wsa[...],
                            preferred_element_type=jnp.float32)
        acc = acc + jnp.dot(hb[...], wsb[...],
                            preferred_element_type=jnp.float32)
        acc = acc + bb[...]
        if relu:
            acc = jnp.maximum(acc, 0.0)
        if final:
            outs[0][...] = acc
        else:
            outs[0][...] = acc[:, :_H]
            outs[1][...] = acc[:, _H:]

    half = pl.BlockSpec((_BN, _H), lambda i: (i, 0))
    in_specs = [
        half, half,
        half, half,
        half, half,
        pl.BlockSpec((_H, _D), lambda i: (0, 0)),
        pl.BlockSpec((_H, _D), lambda i: (0, 0)),
        pl.BlockSpec((_H, _D), lambda i: (0, 0)),
        pl.BlockSpec((_H, _D), lambda i: (0, 0)),
        pl.BlockSpec((1, _D), lambda i: (0, 0)),
    ]
    if final:
        out_specs = pl.BlockSpec((_BN, _D), lambda i: (i, 0))
        out_shape = jax.ShapeDtypeStruct((_N, _D), jnp.float32)
    else:
        out_specs = [half, half]
        out_shape = [jax.ShapeDtypeStruct((_N, _H), jnp.float32),
                     jax.ShapeDtypeStruct((_N, _H), jnp.float32)]
    return pl.pallas_call(body, grid=(_N // _BN,), in_specs=in_specs,
                          out_specs=out_specs, out_shape=out_shape)


_UPDATE_MID = _make_update(True, False)
_UPDATE_FIN = _make_update(False, True)


def _weights(Wn, Ws, b):
    return (Wn[:, :_H].T, Wn[:, _H:].T, Ws[:, :_H].T, Ws[:, _H:].T,
            b.reshape(1, _D))


def kernel(x, edge_index, W_self_0, W_neigh_0, b_0, W_self_1, W_neigh_1,
           b_1, W_self_2, W_neigh_2, b_2):
    edges = jnp.stack(
        [edge_index[0].reshape(_NSUB, _NCH, _K),
         edge_index[1].reshape(_NSUB, _NCH, _K)], axis=2)
    zrows = jnp.zeros((_ROWS, _H), jnp.float32)
    ones40 = jnp.ones((40, _H), jnp.float32)
    dst3 = edge_index[1].reshape(2 * _NSUB, _E // 2 // _NSUB // 40, 40)

    deg_a, deg_b = _DEG(dst3, zrows, ones40)
    ha, hb = x[:, :_H], x[:, _H:]
    agg_a, agg_b = _AGG(ha, hb, edges, zrows)
    ha, hb = _UPDATE_MID(agg_a, agg_b, deg_a, deg_b, ha, hb,
                         *_weights(W_neigh_0, W_self_0, b_0))
    agg_a, agg_b = _AGG(ha, hb, edges, zrows)
    ha, hb = _UPDATE_MID(agg_a, agg_b, deg_a, deg_b, ha, hb,
                         *_weights(W_neigh_1, W_self_1, b_1))
    agg_a, agg_b = _AGG(ha, hb, edges, zrows)
    return _UPDATE_FIN(agg_a, agg_b, deg_a, deg_b, ha, hb,
                       *_weights(W_neigh_2, W_self_2, b_2))


# speedup vs baseline: 1.0093x; 1.0093x over previous
import functools

import jax
import jax.numpy as jnp
from jax import lax
from jax.experimental import pallas as pl
from jax.experimental.pallas import tpu as pltpu
from jax.experimental.pallas import tpu_sc as plsc

_N = 10000
_E = 160000
_D = 256
_H = 128
_NSUB = 16
_K = 80
_EPS = _E // _NSUB
_NCH = _EPS // _K
_ROWS = 624
_EXTRA = _N - _NSUB * _ROWS

_MESH = plsc.VectorSubcoreMesh(core_axis_name="c", subcore_axis_name="s")


def _zero_slices(zsrc, dst_spmem, rbase, s):
    nz = zsrc.shape[0]
    full, tail = divmod(_ROWS, nz)
    for j in range(full):
        pltpu.sync_copy(zsrc, dst_spmem.at[pl.ds(rbase + j * nz, nz)])
    if tail:
        pltpu.sync_copy(zsrc.at[pl.ds(0, tail)],
                        dst_spmem.at[pl.ds(rbase + full * nz, tail)])

    @pl.when(s == _NSUB - 1)
    def _():
        pltpu.sync_copy(zsrc.at[pl.ds(0, _EXTRA)],
                        dst_spmem.at[pl.ds(_NSUB * _ROWS, _EXTRA)])


def _copy_out_slices(acc_spmem, bounce, out_ref, rbase, s):
    nz = bounce.shape[0]
    full, tail = divmod(_ROWS, nz)
    sizes = [nz] * full + ([tail] if tail else [])
    for j, sz in enumerate(sizes):
        r0 = rbase + j * nz
        pltpu.sync_copy(acc_spmem.at[pl.ds(r0, sz)], bounce.at[pl.ds(0, sz)])
        pltpu.sync_copy(bounce.at[pl.ds(0, sz)], out_ref.at[pl.ds(r0, sz)])

    @pl.when(s == _NSUB - 1)
    def _():
        r0 = _NSUB * _ROWS
        pltpu.sync_copy(acc_spmem.at[pl.ds(r0, _EXTRA)],
                        bounce.at[pl.ds(0, _EXTRA)])
        pltpu.sync_copy(bounce.at[pl.ds(0, _EXTRA)],
                        out_ref.at[pl.ds(r0, _EXTRA)])


def _make_agg():
    out_type = [
        jax.ShapeDtypeStruct((_N, _H), jnp.float32),
        jax.ShapeDtypeStruct((_N, _H), jnp.float32),
    ]
    scratch_types = [
        pltpu.VMEM((2, _K), jnp.int32),
        pltpu.VMEM((2, _K), jnp.int32),
        pltpu.VMEM((2, _K), jnp.int32),
        pltpu.VMEM((2, _K), jnp.int32),
        pltpu.VMEM((_K, _H), jnp.float32),
        pltpu.VMEM((_K, _H), jnp.float32),
        pltpu.VMEM_SHARED((_N, _H), jnp.float32),
        pltpu.SemaphoreType.DMA,
        pltpu.SemaphoreType.DMA,
        pltpu.SemaphoreType.DMA,
        pltpu.SemaphoreType.DMA,
        pltpu.SemaphoreType.DMA,
        pltpu.SemaphoreType.DMA,
        pltpu.SemaphoreType.DMA,
        pltpu.SemaphoreType.DMA,
    ]

    @functools.partial(pl.kernel, mesh=_MESH, out_type=out_type,
                       scratch_types=scratch_types)
    def agg(ha, hb, edges, zrows, out_a, out_b,
            ib0, ib1, ib2, ib3, st0, st1, acc,
            semi0, semi1, semi2, semi3, semg0, semg1, sems0, sems1):
        c = lax.axis_index("c")
        s = lax.axis_index("s")
        rbase = s * _ROWS
        my_edges = edges.at[s]

        pltpu.sync_copy(zrows, st0)
        _zero_slices(st0, acc, rbase, s)

        plsc.subcore_barrier()

        def run(h):
            ibs = (ib0, ib1, ib2, ib3)
            semis = (semi0, semi1, semi2, semi3)
            sts = (st0, st1)
            semgs = (semg0, semg1)
            semss = (sems0, sems1)

            def idx_load(i, q):
                pltpu.async_copy(my_edges.at[i], ibs[q], semis[q])

            def idx_wait(q):
                pltpu.make_async_copy(my_edges.at[0], ibs[q],
                                      semis[q]).wait()

            def gather(q, p):
                pltpu.async_copy(h.at[ibs[q].at[0]], sts[p], semgs[p])

            def gather_wait(p):
                pltpu.make_async_copy(h.at[ibs[0].at[0]], sts[p],
                                      semgs[p]).wait()

            def scat_start(q, p):
                pltpu.make_async_copy(sts[p], acc.at[ibs[q].at[1]],
                                      semss[p]).start(add=True)

            def scat_wait(p):
                pltpu.make_async_copy(sts[p], acc.at[ibs[0].at[1]],
                                      semss[p]).wait()

            idx_load(0, 0)
            idx_load(1, 1)
            idx_load(2, 2)
            idx_load(3, 3)
            idx_wait(0)
            gather(0, 0)
            idx_wait(1)
            gather(1, 1)

            def group(j, carry):
                a = 4 * j
                gather_wait(0)
                scat_start(0, 0)
                gather_wait(1)
                scat_start(1, 1)
                scat_wait(0)

                @pl.when(a + 4 < _NCH)
                def _():
                    idx_load(a + 4, 0)

                idx_wait(2)
                gather(2, 0)
                scat_wait(1)

                @pl.when(a + 5 < _NCH)
                def _():
                    idx_load(a + 5, 1)

                idx_wait(3)
                gather(3, 1)
                gather_wait(0)
                scat_start(2, 0)
                gather_wait(1)
                scat_start(3, 1)
                scat_wait(0)

                @pl.when(a + 6 < _NCH)
                def _():
                    idx_load(a + 6, 2)

                @pl.when(a + 4 < _NCH)
                def _():
                    idx_wait(0)
                    gather(0, 0)

                scat_wait(1)

                @pl.when(a + 7 < _NCH)
                def _():
                    idx_load(a + 7, 3)

                @pl.when(a + 5 < _NCH)
                def _():
                    idx_wait(1)
                    gather(1, 1)

                return carry

            lax.fori_loop(0, _NCH // 4, group, 0)
            gather_wait(0)
            scat_start(0, 0)
            scat_wait(0)

        @pl.when(c == 0)
        def _():
            run(ha)

        @pl.when(c == 1)
        def _():
            run(hb)

        plsc.subcore_barrier()

        @pl.when(c == 0)
        def _():
            _copy_out_slices(acc, st0, out_a, rbase, s)

        @pl.when(c == 1)
        def _():
            _copy_out_slices(acc, st0, out_b, rbase, s)

    return agg


def _make_deg():
    kd = 40
    nchd = _E // 2 // _NSUB // kd
    out_type = [
        jax.ShapeDtypeStruct((_N, _H), jnp.float32),
        jax.ShapeDtypeStruct((_N, _H), jnp.float32),
    ]
    scratch_types = [
        pltpu.VMEM((kd,), jnp.int32),
        pltpu.VMEM((kd,), jnp.int32),
        pltpu.VMEM((kd, _H), jnp.float32),
        pltpu.VMEM_SHARED((_N, _H), jnp.float32),
        pltpu.SemaphoreType.DMA,
        pltpu.SemaphoreType.DMA,
        pltpu.SemaphoreType.DMA,
        pltpu.SemaphoreType.DMA,
    ]

    @functools.partial(pl.kernel, mesh=_MESH, out_type=out_type,
                       scratch_types=scratch_types)
    def deg(dst3, zrows, ones_h, deg_a, deg_b, ib0, ib1, st, dacc,
            semi0, semi1, sems0, sems1):
        c = lax.axis_index("c")
        s = lax.axis_index("s")
        rbase = s * _ROWS
        w = c * _NSUB + s
        my_dst = dst3.at[w]

        pltpu.sync_copy(zrows, st)
        _zero_slices(st, dacc, rbase, s)
        pltpu.sync_copy(ones_h, st)
        plsc.subcore_barrier()

        def idx_load(i, ib, sem):
            pltpu.async_copy(my_dst.at[i], ib, sem)

        def idx_wait(ib, sem):
            pltpu.make_async_copy(my_dst.at[0], ib, sem).wait()

        def scat_start(ib, sem):
            pltpu.make_async_copy(st, dacc.at[ib], sem).start(add=True)

        def scat_wait(ib, sem):
            pltpu.make_async_copy(st, dacc.at[ib], sem).wait()

        pltpu.sync_copy(my_dst.at[0], ib0)
        idx_load(1, ib1, semi1)
        scat_start(ib0, sems0)

        def step(j, carry):
            i0 = 2 * j
            idx_wait(ib1, semi1)
            scat_start(ib1, sems1)
            scat_wait(ib0, sems0)
            idx_load(i0 + 2, ib0, semi0)
            idx_wait(ib0, semi0)
            scat_start(ib0, sems0)
            scat_wait(ib1, sems1)

            @pl.when(i0 + 3 < nchd)
            def _():
                idx_load(i0 + 3, ib1, semi1)

            return carry

        lax.fori_loop(0, (nchd - 1) // 2, step, 0)
        scat_wait(ib0, sems0)

        plsc.subcore_barrier()

        @pl.when(c == 0)
        def _():
            _copy_out_slices(dacc, st, deg_a, rbase, s)

        @pl.when(c == 1)
        def _():
            _copy_out_slices(dacc, st, deg_b, rbase, s)

    return deg


_AGG = _make_agg()
_DEG = _make_deg()

_BN = 1000


def _make_update(relu, final):
    def body(aa, ab, dga, dgb, ha, hb, wna, wnb, wsa, wsb, bb, *outs):
        degc = dga[...][:, 0:1] + dgb[...][:, 0:1]
        inv = 1.0 / jnp.maximum(degc, 1.0)
        acc = jnp.dot(aa[...] * inv, wna[...],
                      preferred_element_type=jnp.float32)
        acc = acc + jnp.dot(ab[...] * inv, wnb[...],
                            preferred_element_type=jnp.float32)
        acc = acc + jnp.dot(ha[...], wsa[...],
                            preferred_element_type=jnp.float32)
        acc = acc + jnp.dot(hb[...], wsb[...],
                            preferred_element_type=jnp.float32)
        acc = acc + bb[...]
        if relu:
            acc = jnp.maximum(acc, 0.0)
        if final:
            outs[0][...] = acc
        else:
            outs[0][...] = acc[:, :_H]
            outs[1][...] = acc[:, _H:]

    half = pl.BlockSpec((_BN, _H), lambda i: (i, 0))
    in_specs = [
        half, half,
        half, half,
        half, half,
        pl.BlockSpec((_H, _D), lambda i: (0, 0)),
        pl.BlockSpec((_H, _D), lambda i: (0, 0)),
        pl.BlockSpec((_H, _D), lambda i: (0, 0)),
        pl.BlockSpec((_H, _D), lambda i: (0, 0)),
        pl.BlockSpec((1, _D), lambda i: (0, 0)),
    ]
    if final:
        out_specs = pl.BlockSpec((_BN, _D), lambda i: (i, 0))
        out_shape = jax.ShapeDtypeStruct((_N, _D), jnp.float32)
    else:
        out_specs = [half, half]
        out_shape = [jax.ShapeDtypeStruct((_N, _H), jnp.float32),
                     jax.ShapeDtypeStruct((_N, _H), jnp.float32)]
    return pl.pallas_call(body, grid=(_N // _BN,), in_specs=in_specs,
                          out_specs=out_specs, out_shape=out_shape)


_UPDATE_MID = _make_update(True, False)
_UPDATE_FIN = _make_update(False, True)


def _weights(Wn, Ws, b):
    return (Wn[:, :_H].T, Wn[:, _H:].T, Ws[:, :_H].T, Ws[:, _H:].T,
            b.reshape(1, _D))


def kernel(x, edge_index, W_self_0, W_neigh_0, b_0, W_self_1, W_neigh_1,
           b_1, W_self_2, W_neigh_2, b_2):
    edges = jnp.stack(
        [edge_index[0].reshape(_NSUB, _NCH, _K),
         edge_index[1].reshape(_NSUB, _NCH, _K)], axis=2)
    zrows = jnp.zeros((_K, _H), jnp.float32)
    z40 = jnp.zeros((40, _H), jnp.float32)
    ones40 = jnp.ones((40, _H), jnp.float32)
    dst3 = edge_index[1].reshape(2 * _NSUB, _E // 2 // _NSUB // 40, 40)

    deg_a, deg_b = _DEG(dst3, z40, ones40)
    ha, hb = x[:, :_H], x[:, _H:]
    agg_a, agg_b = _AGG(ha, hb, edges, zrows)
    ha, hb = _UPDATE_MID(agg_a, agg_b, deg_a, deg_b, ha, hb,
                         *_weights(W_neigh_0, W_self_0, b_0))
    agg_a, agg_b = _AGG(ha, hb, edges, zrows)
    ha, hb = _UPDATE_MID(agg_a, agg_b, deg_a, deg_b, ha, hb,
                         *_weights(W_neigh_1, W_self_1, b_1))
    agg_a, agg_b = _AGG(ha, hb, edges, zrows)
    return _UPDATE_FIN(agg_a, agg_b, deg_a, deg_b, ha, hb,
                       *_weights(W_neigh_2, W_self_2, b_2))


# speedup vs baseline: 1.0306x; 1.0211x over previous
import functools

import jax
import jax.numpy as jnp
from jax import lax
from jax.experimental import pallas as pl
from jax.experimental.pallas import tpu as pltpu
from jax.experimental.pallas import tpu_sc as plsc

_N = 10000
_E = 160000
_D = 256
_H = 128
_NSUB = 16
_K = 80
_EPS = _E // _NSUB
_NCH = _EPS // _K
_ROWS = 624
_EXTRA = _N - _NSUB * _ROWS

_MESH = plsc.VectorSubcoreMesh(core_axis_name="c", subcore_axis_name="s")


def _zero_slices(zsrc, dst_spmem, rbase, s):
    nz = zsrc.shape[0]
    full, tail = divmod(_ROWS, nz)
    for j in range(full):
        pltpu.sync_copy(zsrc, dst_spmem.at[pl.ds(rbase + j * nz, nz)])
    if tail:
        pltpu.sync_copy(zsrc.at[pl.ds(0, tail)],
                        dst_spmem.at[pl.ds(rbase + full * nz, tail)])

    @pl.when(s == _NSUB - 1)
    def _():
        pltpu.sync_copy(zsrc.at[pl.ds(0, _EXTRA)],
                        dst_spmem.at[pl.ds(_NSUB * _ROWS, _EXTRA)])


def _copy_out_slices(acc_spmem, bounce, out_ref, rbase, s):
    nz = bounce.shape[0]
    full, tail = divmod(_ROWS, nz)
    sizes = [nz] * full + ([tail] if tail else [])
    for j, sz in enumerate(sizes):
        r0 = rbase + j * nz
        pltpu.sync_copy(acc_spmem.at[pl.ds(r0, sz)], bounce.at[pl.ds(0, sz)])
        pltpu.sync_copy(bounce.at[pl.ds(0, sz)], out_ref.at[pl.ds(r0, sz)])

    @pl.when(s == _NSUB - 1)
    def _():
        r0 = _NSUB * _ROWS
        pltpu.sync_copy(acc_spmem.at[pl.ds(r0, _EXTRA)],
                        bounce.at[pl.ds(0, _EXTRA)])
        pltpu.sync_copy(bounce.at[pl.ds(0, _EXTRA)],
                        out_ref.at[pl.ds(r0, _EXTRA)])


def _make_agg():
    out_type = [
        jax.ShapeDtypeStruct((_N, _H), jnp.float32),
        jax.ShapeDtypeStruct((_N, _H), jnp.float32),
    ]
    scratch_types = [
        pltpu.VMEM((2, _K), jnp.int32),
        pltpu.VMEM((2, _K), jnp.int32),
        pltpu.VMEM((2, _K), jnp.int32),
        pltpu.VMEM((2, _K), jnp.int32),
        pltpu.VMEM((_K, _H), jnp.float32),
        pltpu.VMEM((_K, _H), jnp.float32),
        pltpu.VMEM_SHARED((_N, _H), jnp.float32),
        pltpu.SemaphoreType.DMA,
        pltpu.SemaphoreType.DMA,
        pltpu.SemaphoreType.DMA,
        pltpu.SemaphoreType.DMA,
        pltpu.SemaphoreType.DMA,
        pltpu.SemaphoreType.DMA,
        pltpu.SemaphoreType.DMA,
        pltpu.SemaphoreType.DMA,
    ]

    @functools.partial(pl.kernel, mesh=_MESH, out_type=out_type,
                       scratch_types=scratch_types)
    def agg(ha, hb, edges, zrows, out_a, out_b,
            ib0, ib1, ib2, ib3, st0, st1, acc,
            semi0, semi1, semi2, semi3, semg0, semg1, sems0, sems1):
        c = lax.axis_index("c")
        s = lax.axis_index("s")
        rbase = s * _ROWS
        my_edges = edges.at[s]

        pltpu.sync_copy(zrows, st0)
        _zero_slices(st0, acc, rbase, s)

        plsc.subcore_barrier()

        def run(h):
            ibs = (ib0, ib1, ib2, ib3)
            semis = (semi0, semi1, semi2, semi3)
            sts = (st0, st1)
            semgs = (semg0, semg1)
            semss = (sems0, sems1)

            def idx_load(i, q):
                pltpu.async_copy(my_edges.at[i], ibs[q], semis[q])

            def idx_wait(q):
                pltpu.make_async_copy(my_edges.at[0], ibs[q],
                                      semis[q]).wait()

            def gather(q, p):
                pltpu.async_copy(h.at[ibs[q].at[0]], sts[p], semgs[p])

            def gather_wait(p):
                pltpu.make_async_copy(h.at[ibs[0].at[0]], sts[p],
                                      semgs[p]).wait()

            def scat_start(q, p):
                pltpu.make_async_copy(sts[p], acc.at[ibs[q].at[1]],
                                      semss[p]).start(add=True)

            def scat_wait(p):
                pltpu.make_async_copy(sts[p], acc.at[ibs[0].at[1]],
                                      semss[p]).wait()

            idx_load(0, 0)
            idx_load(1, 1)
            idx_load(2, 2)
            idx_load(3, 3)
            idx_wait(0)
            gather(0, 0)
            idx_wait(1)
            gather(1, 1)

            def group(j, carry):
                a = 4 * j
                gather_wait(0)
                scat_start(0, 0)
                gather_wait(1)
                scat_start(1, 1)
                scat_wait(0)

                @pl.when(a + 4 < _NCH)
                def _():
                    idx_load(a + 4, 0)

                idx_wait(2)
                gather(2, 0)
                scat_wait(1)

                @pl.when(a + 5 < _NCH)
                def _():
                    idx_load(a + 5, 1)

                idx_wait(3)
                gather(3, 1)
                gather_wait(0)
                scat_start(2, 0)
                gather_wait(1)
                scat_start(3, 1)
                scat_wait(0)

                @pl.when(a + 6 < _NCH)
                def _():
                    idx_load(a + 6, 2)

                @pl.when(a + 4 < _NCH)
                def _():
                    idx_wait(0)
                    gather(0, 0)

                scat_wait(1)

                @pl.when(a + 7 < _NCH)
                def _():
                    idx_load(a + 7, 3)

                @pl.when(a + 5 < _NCH)
                def _():
                    idx_wait(1)
                    gather(1, 1)

                return carry

            lax.fori_loop(0, _NCH // 4, group, 0)
            gather_wait(0)
            scat_start(0, 0)
            scat_wait(0)

        @pl.when(c == 0)
        def _():
            run(ha)

        @pl.when(c == 1)
        def _():
            run(hb)

        plsc.subcore_barrier()

        @pl.when(c == 0)
        def _():
            _copy_out_slices(acc, st0, out_a, rbase, s)

        @pl.when(c == 1)
        def _():
            _copy_out_slices(acc, st0, out_b, rbase, s)

    return agg


def _make_deg():
    kd = 40
    nchd = _E // 2 // _NSUB // kd
    out_type = [
        jax.ShapeDtypeStruct((_N, _H), jnp.float32),
        jax.ShapeDtypeStruct((_N, _H), jnp.float32),
    ]
    scratch_types = [
        pltpu.VMEM((kd,), jnp.int32),
        pltpu.VMEM((kd,), jnp.int32),
        pltpu.VMEM((kd, _H), jnp.float32),
        pltpu.VMEM_SHARED((_N, _H), jnp.float32),
        pltpu.SemaphoreType.DMA,
        pltpu.SemaphoreType.DMA,
        pltpu.SemaphoreType.DMA,
        pltpu.SemaphoreType.DMA,
    ]

    @functools.partial(pl.kernel, mesh=_MESH, out_type=out_type,
                       scratch_types=scratch_types)
    def deg(dst3, zrows, ones_h, deg_a, deg_b, ib0, ib1, st, dacc,
            semi0, semi1, sems0, sems1):
        c = lax.axis_index("c")
        s = lax.axis_index("s")
        rbase = s * _ROWS
        w = c * _NSUB + s
        my_dst = dst3.at[w]

        pltpu.sync_copy(zrows, st)
        _zero_slices(st, dacc, rbase, s)
        pltpu.sync_copy(ones_h, st)
        plsc.subcore_barrier()

        def idx_load(i, ib, sem):
            pltpu.async_copy(my_dst.at[i], ib, sem)

        def idx_wait(ib, sem):
            pltpu.make_async_copy(my_dst.at[0], ib, sem).wait()

        def scat(ib):
            pltpu.sync_copy(st, dacc.at[ib], add=True)

        pltpu.sync_copy(my_dst.at[0], ib0)
        idx_load(1, ib1, semi1)

        def step(j, carry):
            i0 = 2 * j
            scat(ib0)
            idx_load(i0 + 2, ib0, semi0)
            idx_wait(ib1, semi1)
            scat(ib1)

            @pl.when(i0 + 3 < nchd)
            def _():
                idx_load(i0 + 3, ib1, semi1)

            idx_wait(ib0, semi0)
            return carry

        lax.fori_loop(0, (nchd - 1) // 2, step, 0)
        scat(ib0)

        plsc.subcore_barrier()

        @pl.when(c == 0)
        def _():
            _copy_out_slices(dacc, st, deg_a, rbase, s)

        @pl.when(c == 1)
        def _():
            _copy_out_slices(dacc, st, deg_b, rbase, s)

    return deg


_AGG = _make_agg()
_DEG = _make_deg()

_BN = 1000


def _make_update(relu, final):
    def body(aa, ab, dga, dgb, ha, hb, wna, wnb, wsa, wsb, bb, *outs):
        degc = dga[...][:, 0:1] + dgb[...][:, 0:1]
        inv = 1.0 / jnp.maximum(degc, 1.0)
        acc = jnp.dot(aa[...] * inv, wna[...],
                      preferred_element_type=jnp.float32)
        acc = acc + jnp.dot(ab[...] * inv, wnb[...],
                            preferred_element_type=jnp.float32)
        acc = acc + jnp.dot(ha[...], wsa[...],
                            preferred_element_type=jnp.float32)
        acc = acc + jnp.dot(hb[...], wsb[...],
                            preferred_element_type=jnp.float32)
        acc = acc + bb[...]
        if relu:
            acc = jnp.maximum(acc, 0.0)
        if final:
            outs[0][...] = acc
        else:
            outs[0][...] = acc[:, :_H]
            outs[1][...] = acc[:, _H:]

    half = pl.BlockSpec((_BN, _H), lambda i: (i, 0))
    in_specs = [
        half, half,
        half, half,
        half, half,
        pl.BlockSpec((_H, _D), lambda i: (0, 0)),
        pl.BlockSpec((_H, _D), lambda i: (0, 0)),
        pl.BlockSpec((_H, _D), lambda i: (0, 0)),
        pl.BlockSpec((_H, _D), lambda i: (0, 0)),
        pl.BlockSpec((1, _D), lambda i: (0, 0)),
    ]
    if final:
        out_specs = pl.BlockSpec((_BN, _D), lambda i: (i, 0))
        out_shape = jax.ShapeDtypeStruct((_N, _D), jnp.float32)
    else:
        out_specs = [half, half]
        out_shape = [jax.ShapeDtypeStruct((_N, _H), jnp.float32),
                     jax.ShapeDtypeStruct((_N, _H), jnp.float32)]
    return pl.pallas_call(body, grid=(_N // _BN,), in_specs=in_specs,
                          out_specs=out_specs, out_shape=out_shape)


_UPDATE_MID = _make_update(True, False)
_UPDATE_FIN = _make_update(False, True)


def _weights(Wn, Ws, b):
    return (Wn[:, :_H].T, Wn[:, _H:].T, Ws[:, :_H].T, Ws[:, _H:].T,
            b.reshape(1, _D))


def kernel(x, edge_index, W_self_0, W_neigh_0, b_0, W_self_1, W_neigh_1,
           b_1, W_self_2, W_neigh_2, b_2):
    edges = jnp.stack(
        [edge_index[0].reshape(_NSUB, _NCH, _K),
         edge_index[1].reshape(_NSUB, _NCH, _K)], axis=2)
    zrows = jnp.zeros((_K, _H), jnp.float32)
    z40 = jnp.zeros((40, _H), jnp.float32)
    ones40 = jnp.ones((40, _H), jnp.float32)
    dst3 = edge_index[1].reshape(2 * _NSUB, _E // 2 // _NSUB // 40, 40)

    deg_a, deg_b = _DEG(dst3, z40, ones40)
    ha, hb = x[:, :_H], x[:, _H:]
    agg_a, agg_b = _AGG(ha, hb, edges, zrows)
    ha, hb = _UPDATE_MID(agg_a, agg_b, deg_a, deg_b, ha, hb,
                         *_weights(W_neigh_0, W_self_0, b_0))
    agg_a, agg_b = _AGG(ha, hb, edges, zrows)
    ha, hb = _UPDATE_MID(agg_a, agg_b, deg_a, deg_b, ha, hb,
                         *_weights(W_neigh_1, W_self_1, b_1))
    agg_a, agg_b = _AGG(ha, hb, edges, zrows)
    return _UPDATE_FIN(agg_a, agg_b, deg_a, deg_b, ha, hb,
                       *_weights(W_neigh_2, W_self_2, b_2))


# speedup vs baseline: 1.0316x; 1.0010x over previous
import functools

import jax
import jax.numpy as jnp
from jax import lax
from jax.experimental import pallas as pl
from jax.experimental.pallas import tpu as pltpu
from jax.experimental.pallas import tpu_sc as plsc

_N = 10000
_E = 160000
_D = 256
_H = 128
_NSUB = 16
_K = 80
_EPS = _E // _NSUB
_NCH = _EPS // _K
_ROWS = 624
_EXTRA = _N - _NSUB * _ROWS

_MESH = plsc.VectorSubcoreMesh(core_axis_name="c", subcore_axis_name="s")


def _zero_slices(zsrc, dst_spmem, rbase, s):
    nz = zsrc.shape[0]
    full, tail = divmod(_ROWS, nz)
    for j in range(full):
        pltpu.sync_copy(zsrc, dst_spmem.at[pl.ds(rbase + j * nz, nz)])
    if tail:
        pltpu.sync_copy(zsrc.at[pl.ds(0, tail)],
                        dst_spmem.at[pl.ds(rbase + full * nz, tail)])

    @pl.when(s == _NSUB - 1)
    def _():
        pltpu.sync_copy(zsrc.at[pl.ds(0, _EXTRA)],
                        dst_spmem.at[pl.ds(_NSUB * _ROWS, _EXTRA)])


def _copy_out_slices(acc_spmem, bounce, out_ref, rbase, s):
    nz = bounce.shape[0]
    full, tail = divmod(_ROWS, nz)
    sizes = [nz] * full + ([tail] if tail else [])
    for j, sz in enumerate(sizes):
        r0 = rbase + j * nz
        pltpu.sync_copy(acc_spmem.at[pl.ds(r0, sz)], bounce.at[pl.ds(0, sz)])
        pltpu.sync_copy(bounce.at[pl.ds(0, sz)], out_ref.at[pl.ds(r0, sz)])

    @pl.when(s == _NSUB - 1)
    def _():
        r0 = _NSUB * _ROWS
        pltpu.sync_copy(acc_spmem.at[pl.ds(r0, _EXTRA)],
                        bounce.at[pl.ds(0, _EXTRA)])
        pltpu.sync_copy(bounce.at[pl.ds(0, _EXTRA)],
                        out_ref.at[pl.ds(r0, _EXTRA)])


def _make_agg():
    out_type = [
        jax.ShapeDtypeStruct((_N, _H), jnp.float32),
        jax.ShapeDtypeStruct((_N, _H), jnp.float32),
    ]
    scratch_types = [
        pltpu.VMEM((2, _K), jnp.int32),
        pltpu.VMEM((2, _K), jnp.int32),
        pltpu.VMEM((2, _K), jnp.int32),
        pltpu.VMEM((2, _K), jnp.int32),
        pltpu.VMEM((_K, _H), jnp.float32),
        pltpu.VMEM((_K, _H), jnp.float32),
        pltpu.VMEM_SHARED((_N, _H), jnp.float32),
        pltpu.SemaphoreType.DMA,
        pltpu.SemaphoreType.DMA,
        pltpu.SemaphoreType.DMA,
        pltpu.SemaphoreType.DMA,
        pltpu.SemaphoreType.DMA,
        pltpu.SemaphoreType.DMA,
        pltpu.SemaphoreType.DMA,
        pltpu.SemaphoreType.DMA,
    ]

    @functools.partial(pl.kernel, mesh=_MESH, out_type=out_type,
                       scratch_types=scratch_types)
    def agg(ha, hb, edges, zrows, out_a, out_b,
            ib0, ib1, ib2, ib3, st0, st1, acc,
            semi0, semi1, semi2, semi3, semg0, semg1, sems0, sems1):
        c = lax.axis_index("c")
        s = lax.axis_index("s")
        rbase = s * _ROWS
        my_edges = edges.at[s]

        pltpu.sync_copy(zrows, st0)
        _zero_slices(st0, acc, rbase, s)

        plsc.subcore_barrier()

        def run(h):
            ibs = (ib0, ib1, ib2, ib3)
            semis = (semi0, semi1, semi2, semi3)
            sts = (st0, st1)
            semgs = (semg0, semg1)
            semss = (sems0, sems1)

            def idx_load(i, q):
                pltpu.async_copy(my_edges.at[i], ibs[q], semis[q])

            def idx_wait(q):
                pltpu.make_async_copy(my_edges.at[0], ibs[q],
                                      semis[q]).wait()

            def gather(q, p):
                pltpu.async_copy(h.at[ibs[q].at[0]], sts[p], semgs[p])

            def gather_wait(p):
                pltpu.make_async_copy(h.at[ibs[0].at[0]], sts[p],
                                      semgs[p]).wait()

            def scat_start(q, p):
                pltpu.make_async_copy(sts[p], acc.at[ibs[q].at[1]],
                                      semss[p]).start(add=True)

            def scat_wait(p):
                pltpu.make_async_copy(sts[p], acc.at[ibs[0].at[1]],
                                      semss[p]).wait()

            idx_load(0, 0)
            idx_load(1, 1)
            idx_load(2, 2)
            idx_load(3, 3)
            idx_wait(0)
            gather(0, 0)
            idx_wait(1)
            gather(1, 1)

            def group(j, carry):
                a = 4 * j
                gather_wait(0)
                scat_start(0, 0)
                gather_wait(1)
                scat_start(1, 1)
                scat_wait(0)

                @pl.when(a + 4 < _NCH)
                def _():
                    idx_load(a + 4, 0)

                idx_wait(2)
                gather(2, 0)
                scat_wait(1)

                @pl.when(a + 5 < _NCH)
                def _():
                    idx_load(a + 5, 1)

                idx_wait(3)
                gather(3, 1)
                gather_wait(0)
                scat_start(2, 0)
                gather_wait(1)
                scat_start(3, 1)
                scat_wait(0)

                @pl.when(a + 6 < _NCH)
                def _():
                    idx_load(a + 6, 2)

                @pl.when(a + 4 < _NCH)
                def _():
                    idx_wait(0)
                    gather(0, 0)

                scat_wait(1)

                @pl.when(a + 7 < _NCH)
                def _():
                    idx_load(a + 7, 3)

                @pl.when(a + 5 < _NCH)
                def _():
                    idx_wait(1)
                    gather(1, 1)

                return carry

            lax.fori_loop(0, _NCH // 4, group, 0)
            gather_wait(0)
            scat_start(0, 0)
            scat_wait(0)

        @pl.when(c == 0)
        def _():
            run(ha)

        @pl.when(c == 1)
        def _():
            run(hb)

        plsc.subcore_barrier()

        @pl.when(c == 0)
        def _():
            _copy_out_slices(acc, st0, out_a, rbase, s)

        @pl.when(c == 1)
        def _():
            _copy_out_slices(acc, st0, out_b, rbase, s)

    return agg


def _make_deg():
    kd = 40
    nchd = _E // 2 // _NSUB // kd
    out_type = [
        jax.ShapeDtypeStruct((_N, _H), jnp.float32),
        jax.ShapeDtypeStruct((_N, _H), jnp.float32),
    ]
    scratch_types = [
        pltpu.VMEM((kd,), jnp.int32),
        pltpu.VMEM((kd,), jnp.int32),
        pltpu.VMEM((kd, _H), jnp.float32),
        pltpu.VMEM_SHARED((_N, _H), jnp.float32),
        pltpu.SemaphoreType.DMA,
        pltpu.SemaphoreType.DMA,
    ]

    @functools.partial(pl.kernel, mesh=_MESH, out_type=out_type,
                       scratch_types=scratch_types)
    def deg(dst3, zrows, ones_h, deg_a, deg_b, ib0, ib1, st, dacc,
            semi0, semi1):
        c = lax.axis_index("c")
        s = lax.axis_index("s")
        rbase = s * _ROWS
        w = c * _NSUB + s
        my_dst = dst3.at[w]

        pltpu.sync_copy(zrows, st)
        _zero_slices(st, dacc, rbase, s)
        pltpu.sync_copy(ones_h, st)
        plsc.subcore_barrier()

        def idx_load(i, ib, sem):
            pltpu.async_copy(my_dst.at[i], ib, sem)

        def idx_wait(ib, sem):
            pltpu.make_async_copy(my_dst.at[0], ib, sem).wait()

        def scat(ib):
            pltpu.sync_copy(st, dacc.at[ib], add=True)

        pltpu.sync_copy(my_dst.at[0], ib0)
        idx_load(1, ib1, semi1)

        def step(j, carry):
            i0 = 2 * j
            scat(ib0)
            idx_load(i0 + 2, ib0, semi0)
            idx_wait(ib1, semi1)
            scat(ib1)

            @pl.when(i0 + 3 < nchd)
            def _():
                idx_load(i0 + 3, ib1, semi1)

            idx_wait(ib0, semi0)
            return carry

        lax.fori_loop(0, (nchd - 1) // 2, step, 0)
        scat(ib0)

        plsc.subcore_barrier()

        @pl.when(c == 0)
        def _():
            _copy_out_slices(dacc, st, deg_a, rbase, s)

        @pl.when(c == 1)
        def _():
            _copy_out_slices(dacc, st, deg_b, rbase, s)

    return deg


_AGG = _make_agg()
_DEG = _make_deg()

_BN = 1000


def _make_update(relu, final):
    def body(aa, ab, dga, dgb, ha, hb, wna, wnb, wsa, wsb, bb, *outs):
        degc = dga[...][:, 0:1] + dgb[...][:, 0:1]
        inv = 1.0 / jnp.maximum(degc, 1.0)
        acc = jnp.dot(aa[...] * inv, wna[...],
                      preferred_element_type=jnp.float32)
        acc = acc + jnp.dot(ab[...] * inv, wnb[...],
                            preferred_element_type=jnp.float32)
        acc = acc + jnp.dot(ha[...], wsa[...],
                            preferred_element_type=jnp.float32)
        acc = acc + jnp.dot(hb[...], wsb[...],
                            preferred_element_type=jnp.float32)
        acc = acc + bb[...]
        if relu:
            acc = jnp.maximum(acc, 0.0)
        if final:
            outs[0][...] = acc
        else:
            outs[0][...] = acc[:, :_H]
            outs[1][...] = acc[:, _H:]

    half = pl.BlockSpec((_BN, _H), lambda i: (i, 0))
    in_specs = [
        half, half,
        half, half,
        half, half,
        pl.BlockSpec((_H, _D), lambda i: (0, 0)),
        pl.BlockSpec((_H, _D), lambda i: (0, 0)),
        pl.BlockSpec((_H, _D), lambda i: (0, 0)),
        pl.BlockSpec((_H, _D), lambda i: (0, 0)),
        pl.BlockSpec((1, _D), lambda i: (0, 0)),
    ]
    if final:
        out_specs = pl.BlockSpec((_BN, _D), lambda i: (i, 0))
        out_shape = jax.ShapeDtypeStruct((_N, _D), jnp.float32)
    else:
        out_specs = [half, half]
        out_shape = [jax.ShapeDtypeStruct((_N, _H), jnp.float32),
                     jax.ShapeDtypeStruct((_N, _H), jnp.float32)]
    return pl.pallas_call(body, grid=(_N // _BN,), in_specs=in_specs,
                          out_specs=out_specs, out_shape=out_shape)


_UPDATE_MID = _make_update(True, False)
_UPDATE_FIN = _make_update(False, True)


def _weights(Wn, Ws, b):
    return (Wn[:, :_H].T, Wn[:, _H:].T, Ws[:, :_H].T, Ws[:, _H:].T,
            b.reshape(1, _D))


def kernel(x, edge_index, W_self_0, W_neigh_0, b_0, W_self_1, W_neigh_1,
           b_1, W_self_2, W_neigh_2, b_2):
    edges = jnp.stack(
        [edge_index[0].reshape(_NSUB, _NCH, _K),
         edge_index[1].reshape(_NSUB, _NCH, _K)], axis=2)
    zrows = jnp.zeros((_K, _H), jnp.float32)
    z40 = jnp.zeros((40, _H), jnp.float32)
    ones40 = jnp.ones((40, _H), jnp.float32)
    dst3 = edge_index[1].reshape(2 * _NSUB, _E // 2 // _NSUB // 40, 40)

    deg_a, deg_b = _DEG(dst3, z40, ones40)
    ha, hb = x[:, :_H], x[:, _H:]
    agg_a, agg_b = _AGG(ha, hb, edges, zrows)
    ha, hb = _UPDATE_MID(agg_a, agg_b, deg_a, deg_b, ha, hb,
                         *_weights(W_neigh_0, W_self_0, b_0))
    agg_a, agg_b = _AGG(ha, hb, edges, zrows)
    ha, hb = _UPDATE_MID(agg_a, agg_b, deg_a, deg_b, ha, hb,
                         *_weights(W_neigh_1, W_self_1, b_1))
    agg_a, agg_b = _AGG(ha, hb, edges, zrows)
    return _UPDATE_FIN(agg_a, agg_b, deg_a, deg_b, ha, hb,
                       *_weights(W_neigh_2, W_self_2, b_2))


# speedup vs baseline: 1.3969x; 1.3541x over previous
import functools

import jax
import jax.numpy as jnp
from jax import lax
from jax.experimental import pallas as pl
from jax.experimental.pallas import tpu as pltpu
from jax.experimental.pallas import tpu_sc as plsc

_N = 10000
_E = 160000
_D = 256
_H = 128
_NSUB = 16
_K = 80
_EPS = _E // _NSUB
_NCH = _EPS // _K
_ROWS = 624
_EXTRA = _N - _NSUB * _ROWS

_MESH = plsc.VectorSubcoreMesh(core_axis_name="c", subcore_axis_name="s")


def _zero_slices(zsrc, dst_spmem, rbase, s):
    nz = zsrc.shape[0]
    full, tail = divmod(_ROWS, nz)
    for j in range(full):
        pltpu.sync_copy(zsrc, dst_spmem.at[pl.ds(rbase + j * nz, nz)])
    if tail:
        pltpu.sync_copy(zsrc.at[pl.ds(0, tail)],
                        dst_spmem.at[pl.ds(rbase + full * nz, tail)])

    @pl.when(s == _NSUB - 1)
    def _():
        pltpu.sync_copy(zsrc.at[pl.ds(0, _EXTRA)],
                        dst_spmem.at[pl.ds(_NSUB * _ROWS, _EXTRA)])


def _copy_out_slices(acc_spmem, bounce, out_ref, rbase, s):
    nz = bounce.shape[0]
    full, tail = divmod(_ROWS, nz)
    sizes = [nz] * full + ([tail] if tail else [])
    for j, sz in enumerate(sizes):
        r0 = rbase + j * nz
        pltpu.sync_copy(acc_spmem.at[pl.ds(r0, sz)], bounce.at[pl.ds(0, sz)])
        pltpu.sync_copy(bounce.at[pl.ds(0, sz)], out_ref.at[pl.ds(r0, sz)])

    @pl.when(s == _NSUB - 1)
    def _():
        r0 = _NSUB * _ROWS
        pltpu.sync_copy(acc_spmem.at[pl.ds(r0, _EXTRA)],
                        bounce.at[pl.ds(0, _EXTRA)])
        pltpu.sync_copy(bounce.at[pl.ds(0, _EXTRA)],
                        out_ref.at[pl.ds(r0, _EXTRA)])


def _make_agg():
    out_type = [
        jax.ShapeDtypeStruct((_N, _H), jnp.float32),
        jax.ShapeDtypeStruct((_N, _H), jnp.float32),
    ]
    scratch_types = [
        pltpu.VMEM((2, _K), jnp.int32),
        pltpu.VMEM((2, _K), jnp.int32),
        pltpu.VMEM((2, _K), jnp.int32),
        pltpu.VMEM((2, _K), jnp.int32),
        pltpu.VMEM((2, _K), jnp.int32),
        pltpu.VMEM((2, _K), jnp.int32),
        pltpu.VMEM((_K, _H), jnp.float32),
        pltpu.VMEM((_K, _H), jnp.float32),
        pltpu.VMEM((_K, _H), jnp.float32),
        pltpu.VMEM_SHARED((_N, _H), jnp.float32),
        pltpu.SemaphoreType.DMA,
        pltpu.SemaphoreType.DMA,
        pltpu.SemaphoreType.DMA,
        pltpu.SemaphoreType.DMA,
        pltpu.SemaphoreType.DMA,
        pltpu.SemaphoreType.DMA,
        pltpu.SemaphoreType.DMA,
        pltpu.SemaphoreType.DMA,
        pltpu.SemaphoreType.DMA,
        pltpu.SemaphoreType.DMA,
        pltpu.SemaphoreType.DMA,
        pltpu.SemaphoreType.DMA,
    ]

    @functools.partial(pl.kernel, mesh=_MESH, out_type=out_type,
                       scratch_types=scratch_types)
    def agg(ha, hb, edges, zrows, out_a, out_b,
            ib0, ib1, ib2, ib3, ib4, ib5, st0, st1, st2, acc,
            semi0, semi1, semi2, semi3, semi4, semi5,
            semg0, semg1, semg2, sems0, sems1, sems2):
        c = lax.axis_index("c")
        s = lax.axis_index("s")
        rbase = s * _ROWS
        my_edges = edges.at[s]

        pltpu.sync_copy(zrows, st0)
        _zero_slices(st0, acc, rbase, s)

        plsc.subcore_barrier()

        def run(h):
            ibs = (ib0, ib1, ib2, ib3, ib4, ib5)
            semis = (semi0, semi1, semi2, semi3, semi4, semi5)
            sts = (st0, st1, st2)
            semgs = (semg0, semg1, semg2)
            semss = (sems0, sems1, sems2)

            def idx_load(i, q):
                pltpu.async_copy(my_edges.at[i], ibs[q], semis[q])

            def idx_wait(q):
                pltpu.make_async_copy(my_edges.at[0], ibs[q],
                                      semis[q]).wait()

            def gather(q, p):
                pltpu.async_copy(h.at[ibs[q].at[0]], sts[p], semgs[p])

            def gather_wait(p):
                pltpu.make_async_copy(h.at[ibs[0].at[0]], sts[p],
                                      semgs[p]).wait()

            def scat_start(q, p):
                pltpu.make_async_copy(sts[p], acc.at[ibs[q].at[1]],
                                      semss[p]).start(add=True)

            def scat_wait(p):
                pltpu.make_async_copy(sts[p], acc.at[ibs[0].at[1]],
                                      semss[p]).wait()

            idx_load(0, 0)
            idx_load(1, 1)
            idx_load(2, 2)

            def steps(j, carry):
                for r in range(6):
                    i = 6 * j + r
                    p = r % 3
                    pm1 = (r - 1) % 3
                    q = r
                    qm1 = (r - 1) % 6
                    qp3 = (r + 3) % 6

                    @pl.when(jnp.logical_and(i - 3 >= 0, i - 3 < _NCH))
                    def _(p=p):
                        scat_wait(p)

                    @pl.when(i + 3 < _NCH)
                    def _(i=i, qp3=qp3):
                        idx_load(i + 3, qp3)

                    @pl.when(i < _NCH)
                    def _(q=q, p=p):
                        idx_wait(q)
                        gather(q, p)

                    @pl.when(jnp.logical_and(i - 1 >= 0, i - 1 < _NCH))
                    def _(qm1=qm1, pm1=pm1):
                        gather_wait(pm1)
                        scat_start(qm1, pm1)
                return carry

            lax.fori_loop(0, (_NCH + 3 + 5) // 6 + 1, steps, 0)

        @pl.when(c == 0)
        def _():
            run(ha)

        @pl.when(c == 1)
        def _():
            run(hb)

        plsc.subcore_barrier()

        @pl.when(c == 0)
        def _():
            _copy_out_slices(acc, st0, out_a, rbase, s)

        @pl.when(c == 1)
        def _():
            _copy_out_slices(acc, st0, out_b, rbase, s)

    return agg


def _make_deg():
    kd = 40
    nchd = _E // 2 // _NSUB // kd
    out_type = [
        jax.ShapeDtypeStruct((_N, _H), jnp.float32),
        jax.ShapeDtypeStruct((_N, _H), jnp.float32),
    ]
    scratch_types = [
        pltpu.VMEM((kd,), jnp.int32),
        pltpu.VMEM((kd,), jnp.int32),
        pltpu.VMEM((kd, _H), jnp.float32),
        pltpu.VMEM_SHARED((_N, _H), jnp.float32),
        pltpu.SemaphoreType.DMA,
        pltpu.SemaphoreType.DMA,
    ]

    @functools.partial(pl.kernel, mesh=_MESH, out_type=out_type,
                       scratch_types=scratch_types)
    def deg(dst3, zrows, ones_h, deg_a, deg_b, ib0, ib1, st, dacc,
            semi0, semi1):
        c = lax.axis_index("c")
        s = lax.axis_index("s")
        rbase = s * _ROWS
        w = c * _NSUB + s
        my_dst = dst3.at[w]

        pltpu.sync_copy(zrows, st)
        _zero_slices(st, dacc, rbase, s)
        pltpu.sync_copy(ones_h, st)
        plsc.subcore_barrier()

        def idx_load(i, ib, sem):
            pltpu.async_copy(my_dst.at[i], ib, sem)

        def idx_wait(ib, sem):
            pltpu.make_async_copy(my_dst.at[0], ib, sem).wait()

        def scat(ib):
            pltpu.sync_copy(st, dacc.at[ib], add=True)

        pltpu.sync_copy(my_dst.at[0], ib0)
        idx_load(1, ib1, semi1)

        def step(j, carry):
            i0 = 2 * j
            scat(ib0)
            idx_load(i0 + 2, ib0, semi0)
            idx_wait(ib1, semi1)
            scat(ib1)

            @pl.when(i0 + 3 < nchd)
            def _():
                idx_load(i0 + 3, ib1, semi1)

            idx_wait(ib0, semi0)
            return carry

        lax.fori_loop(0, (nchd - 1) // 2, step, 0)
        scat(ib0)

        plsc.subcore_barrier()

        @pl.when(c == 0)
        def _():
            _copy_out_slices(dacc, st, deg_a, rbase, s)

        @pl.when(c == 1)
        def _():
            _copy_out_slices(dacc, st, deg_b, rbase, s)

    return deg


_AGG = _make_agg()
_DEG = _make_deg()

_BN = 1000


def _make_update(relu, final):
    def body(aa, ab, dga, dgb, ha, hb, wna, wnb, wsa, wsb, bb, *outs):
        degc = dga[...][:, 0:1] + dgb[...][:, 0:1]
        inv = 1.0 / jnp.maximum(degc, 1.0)
        acc = jnp.dot(aa[...] * inv, wna[...],
                      preferred_element_type=jnp.float32)
        acc = acc + jnp.dot(ab[...] * inv, wnb[...],
                            preferred_element_type=jnp.float32)
        acc = acc + jnp.dot(ha[...], wsa[...],
                            preferred_element_type=jnp.float32)
        acc = acc + jnp.dot(hb[...], wsb[...],
                            preferred_element_type=jnp.float32)
        acc = acc + bb[...]
        if relu:
            acc = jnp.maximum(acc, 0.0)
        if final:
            outs[0][...] = acc
        else:
            outs[0][...] = acc[:, :_H]
            outs[1][...] = acc[:, _H:]

    half = pl.BlockSpec((_BN, _H), lambda i: (i, 0))
    in_specs = [
        half, half,
        half, half,
        half, half,
        pl.BlockSpec((_H, _D), lambda i: (0, 0)),
        pl.BlockSpec((_H, _D), lambda i: (0, 0)),
        pl.BlockSpec((_H, _D), lambda i: (0, 0)),
        pl.BlockSpec((_H, _D), lambda i: (0, 0)),
        pl.BlockSpec((1, _D), lambda i: (0, 0)),
    ]
    if final:
        out_specs = pl.BlockSpec((_BN, _D), lambda i: (i, 0))
        out_shape = jax.ShapeDtypeStruct((_N, _D), jnp.float32)
    else:
        out_specs = [half, half]
        out_shape = [jax.ShapeDtypeStruct((_N, _H), jnp.float32),
                     jax.ShapeDtypeStruct((_N, _H), jnp.float32)]
    return pl.pallas_call(body, grid=(_N // _BN,), in_specs=in_specs,
                          out_specs=out_specs, out_shape=out_shape)


_UPDATE_MID = _make_update(True, False)
_UPDATE_FIN = _make_update(False, True)


def _weights(Wn, Ws, b):
    return (Wn[:, :_H].T, Wn[:, _H:].T, Ws[:, :_H].T, Ws[:, _H:].T,
            b.reshape(1, _D))


def kernel(x, edge_index, W_self_0, W_neigh_0, b_0, W_self_1, W_neigh_1,
           b_1, W_self_2, W_neigh_2, b_2):
    edges = jnp.stack(
        [edge_index[0].reshape(_NSUB, _NCH, _K),
         edge_index[1].reshape(_NSUB, _NCH, _K)], axis=2)
    zrows = jnp.zeros((_K, _H), jnp.float32)
    z40 = jnp.zeros((40, _H), jnp.float32)
    ones40 = jnp.ones((40, _H), jnp.float32)
    dst3 = edge_index[1].reshape(2 * _NSUB, _E // 2 // _NSUB // 40, 40)

    deg_a, deg_b = _DEG(dst3, z40, ones40)
    ha, hb = x[:, :_H], x[:, _H:]
    agg_a, agg_b = _AGG(ha, hb, edges, zrows)
    ha, hb = _UPDATE_MID(agg_a, agg_b, deg_a, deg_b, ha, hb,
                         *_weights(W_neigh_0, W_self_0, b_0))
    agg_a, agg_b = _AGG(ha, hb, edges, zrows)
    ha, hb = _UPDATE_MID(agg_a, agg_b, deg_a, deg_b, ha, hb,
                         *_weights(W_neigh_1, W_self_1, b_1))
    agg_a, agg_b = _AGG(ha, hb, edges, zrows)
    return _UPDATE_FIN(agg_a, agg_b, deg_a, deg_b, ha, hb,
                       *_weights(W_neigh_2, W_self_2, b_2))


# speedup vs baseline: 1.4528x; 1.0400x over previous
import functools

import jax
import jax.numpy as jnp
from jax import lax
from jax.experimental import pallas as pl
from jax.experimental.pallas import tpu as pltpu
from jax.experimental.pallas import tpu_sc as plsc

_N = 10000
_E = 160000
_D = 256
_H = 128
_NSUB = 16
_K = 80
_EPS = _E // _NSUB
_NCH = _EPS // _K
_ROWS = 624
_EXTRA = _N - _NSUB * _ROWS

_MESH = plsc.VectorSubcoreMesh(core_axis_name="c", subcore_axis_name="s")


def _zero_slices(zsrc, dst_spmem, rbase, s):
    nz = zsrc.shape[0]
    full, tail = divmod(_ROWS, nz)
    for j in range(full):
        pltpu.sync_copy(zsrc, dst_spmem.at[pl.ds(rbase + j * nz, nz)])
    if tail:
        pltpu.sync_copy(zsrc.at[pl.ds(0, tail)],
                        dst_spmem.at[pl.ds(rbase + full * nz, tail)])

    @pl.when(s == _NSUB - 1)
    def _():
        pltpu.sync_copy(zsrc.at[pl.ds(0, _EXTRA)],
                        dst_spmem.at[pl.ds(_NSUB * _ROWS, _EXTRA)])


def _copy_out_slices(acc_spmem, bounce, out_ref, rbase, s):
    nz = bounce.shape[0]
    full, tail = divmod(_ROWS, nz)
    sizes = [nz] * full + ([tail] if tail else [])
    for j, sz in enumerate(sizes):
        r0 = rbase + j * nz
        pltpu.sync_copy(acc_spmem.at[pl.ds(r0, sz)], bounce.at[pl.ds(0, sz)])
        pltpu.sync_copy(bounce.at[pl.ds(0, sz)], out_ref.at[pl.ds(r0, sz)])

    @pl.when(s == _NSUB - 1)
    def _():
        r0 = _NSUB * _ROWS
        pltpu.sync_copy(acc_spmem.at[pl.ds(r0, _EXTRA)],
                        bounce.at[pl.ds(0, _EXTRA)])
        pltpu.sync_copy(bounce.at[pl.ds(0, _EXTRA)],
                        out_ref.at[pl.ds(r0, _EXTRA)])


def _make_agg():
    out_type = [
        jax.ShapeDtypeStruct((_N, _H), jnp.float32),
        jax.ShapeDtypeStruct((_N, _H), jnp.float32),
    ]
    scratch_types = [
        pltpu.VMEM((2, _K), jnp.int32),
        pltpu.VMEM((2, _K), jnp.int32),
        pltpu.VMEM((2, _K), jnp.int32),
        pltpu.VMEM((2, _K), jnp.int32),
        pltpu.VMEM((2, _K), jnp.int32),
        pltpu.VMEM((2, _K), jnp.int32),
        pltpu.VMEM((_K, _H), jnp.float32),
        pltpu.VMEM((_K, _H), jnp.float32),
        pltpu.VMEM((_K, _H), jnp.float32),
        pltpu.VMEM_SHARED((_N, _H), jnp.float32),
        pltpu.SemaphoreType.DMA,
        pltpu.SemaphoreType.DMA,
        pltpu.SemaphoreType.DMA,
        pltpu.SemaphoreType.DMA,
        pltpu.SemaphoreType.DMA,
        pltpu.SemaphoreType.DMA,
        pltpu.SemaphoreType.DMA,
        pltpu.SemaphoreType.DMA,
        pltpu.SemaphoreType.DMA,
        pltpu.SemaphoreType.DMA,
        pltpu.SemaphoreType.DMA,
        pltpu.SemaphoreType.DMA,
    ]

    @functools.partial(pl.kernel, mesh=_MESH, out_type=out_type,
                       scratch_types=scratch_types)
    def agg(ha, hb, edges, zrows, out_a, out_b,
            ib0, ib1, ib2, ib3, ib4, ib5, st0, st1, st2, acc,
            semi0, semi1, semi2, semi3, semi4, semi5,
            semg0, semg1, semg2, sems0, sems1, sems2):
        c = lax.axis_index("c")
        s = lax.axis_index("s")
        rbase = s * _ROWS
        my_edges = edges.at[s]

        pltpu.sync_copy(zrows, st0)
        _zero_slices(st0, acc, rbase, s)

        plsc.subcore_barrier()

        def run(h):
            ibs = (ib0, ib1, ib2, ib3, ib4, ib5)
            semis = (semi0, semi1, semi2, semi3, semi4, semi5)
            sts = (st0, st1, st2)
            semgs = (semg0, semg1, semg2)
            semss = (sems0, sems1, sems2)

            def idx_load(i, q):
                pltpu.async_copy(my_edges.at[i], ibs[q], semis[q])

            def idx_wait(q):
                pltpu.make_async_copy(my_edges.at[0], ibs[q],
                                      semis[q]).wait()

            def gather(q, p):
                pltpu.async_copy(h.at[ibs[q].at[0]], sts[p], semgs[p])

            def gather_wait(p):
                pltpu.make_async_copy(h.at[ibs[0].at[0]], sts[p],
                                      semgs[p]).wait()

            def scat_start(q, p):
                pltpu.make_async_copy(sts[p], acc.at[ibs[q].at[1]],
                                      semss[p]).start(add=True)

            def scat_wait(p):
                pltpu.make_async_copy(sts[p], acc.at[ibs[0].at[1]],
                                      semss[p]).wait()

            idx_load(0, 0)
            idx_load(1, 1)
            idx_load(2, 2)

            def steps(j, carry):
                for r in range(6):
                    i = 6 * j + r
                    p = r % 3
                    pm1 = (r - 1) % 3
                    q = r
                    qm1 = (r - 1) % 6
                    qp3 = (r + 3) % 6

                    @pl.when(jnp.logical_and(i - 3 >= 0, i - 3 < _NCH))
                    def _(p=p):
                        scat_wait(p)

                    @pl.when(i + 3 < _NCH)
                    def _(i=i, qp3=qp3):
                        idx_load(i + 3, qp3)

                    @pl.when(i < _NCH)
                    def _(q=q, p=p):
                        idx_wait(q)
                        gather(q, p)

                    @pl.when(jnp.logical_and(i - 1 >= 0, i - 1 < _NCH))
                    def _(qm1=qm1, pm1=pm1):
                        gather_wait(pm1)
                        scat_start(qm1, pm1)
                return carry

            lax.fori_loop(0, (_NCH + 3 + 5) // 6 + 1, steps, 0)

        @pl.when(c == 0)
        def _():
            run(ha)

        @pl.when(c == 1)
        def _():
            run(hb)

        plsc.subcore_barrier()

        @pl.when(c == 0)
        def _():
            _copy_out_slices(acc, st0, out_a, rbase, s)

        @pl.when(c == 1)
        def _():
            _copy_out_slices(acc, st0, out_b, rbase, s)

    return agg


def _make_deg():
    kd = 40
    nchd = _E // 2 // _NSUB // kd
    out_type = [
        jax.ShapeDtypeStruct((_N, _H), jnp.float32),
        jax.ShapeDtypeStruct((_N, _H), jnp.float32),
    ]
    scratch_types = [
        pltpu.VMEM((kd,), jnp.int32),
        pltpu.VMEM((kd,), jnp.int32),
        pltpu.VMEM((kd,), jnp.int32),
        pltpu.VMEM((kd,), jnp.int32),
        pltpu.VMEM((kd, _H), jnp.float32),
        pltpu.VMEM_SHARED((_N, _H), jnp.float32),
        pltpu.SemaphoreType.DMA,
        pltpu.SemaphoreType.DMA,
        pltpu.SemaphoreType.DMA,
        pltpu.SemaphoreType.DMA,
        pltpu.SemaphoreType.DMA,
        pltpu.SemaphoreType.DMA,
    ]

    @functools.partial(pl.kernel, mesh=_MESH, out_type=out_type,
                       scratch_types=scratch_types)
    def deg(dst3, zrows, ones_h, deg_a, deg_b, ib0, ib1, ib2, ib3, st,
            dacc, semi0, semi1, semi2, semi3, sems0, sems1):
        c = lax.axis_index("c")
        s = lax.axis_index("s")
        rbase = s * _ROWS
        w = c * _NSUB + s
        my_dst = dst3.at[w]

        pltpu.sync_copy(zrows, st)
        _zero_slices(st, dacc, rbase, s)
        pltpu.sync_copy(ones_h, st)
        plsc.subcore_barrier()

        ibs = (ib0, ib1, ib2, ib3)
        semis = (semi0, semi1, semi2, semi3)
        semss = (sems0, sems1)

        def idx_load(i, q):
            pltpu.async_copy(my_dst.at[i], ibs[q], semis[q])

        def idx_wait(q):
            pltpu.make_async_copy(my_dst.at[0], ibs[q], semis[q]).wait()

        def scat_start(q, p):
            pltpu.make_async_copy(st, dacc.at[ibs[q]], semss[p]).start(
                add=True)

        def scat_wait(p):
            pltpu.make_async_copy(st, dacc.at[ibs[0]], semss[p]).wait()

        idx_load(0, 0)
        idx_load(1, 1)

        def steps(j, carry):
            for r in range(4):
                i = 4 * j + r
                p = r % 2
                q = r
                qp2 = (r + 2) % 4

                @pl.when(jnp.logical_and(i - 2 >= 0, i - 2 < nchd))
                def _(p=p):
                    scat_wait(p)

                @pl.when(i + 2 < nchd)
                def _(i=i, qp2=qp2):
                    idx_load(i + 2, qp2)

                @pl.when(i < nchd)
                def _(q=q, p=p):
                    idx_wait(q)
                    scat_start(q, p)
            return carry

        lax.fori_loop(0, (nchd + 2 + 3) // 4 + 1, steps, 0)

        plsc.subcore_barrier()

        @pl.when(c == 0)
        def _():
            _copy_out_slices(dacc, st, deg_a, rbase, s)

        @pl.when(c == 1)
        def _():
            _copy_out_slices(dacc, st, deg_b, rbase, s)

    return deg


_AGG = _make_agg()
_DEG = _make_deg()

_BN = 1000


def _make_update(relu, final):
    def body(aa, ab, dga, dgb, ha, hb, wna, wnb, wsa, wsb, bb, *outs):
        degc = dga[...][:, 0:1] + dgb[...][:, 0:1]
        inv = 1.0 / jnp.maximum(degc, 1.0)
        acc = jnp.dot(aa[...] * inv, wna[...],
                      preferred_element_type=jnp.float32)
        acc = acc + jnp.dot(ab[...] * inv, wnb[...],
                            preferred_element_type=jnp.float32)
        acc = acc + jnp.dot(ha[...], wsa[...],
                            preferred_element_type=jnp.float32)
        acc = acc + jnp.dot(hb[...], wsb[...],
                            preferred_element_type=jnp.float32)
        acc = acc + bb[...]
        if relu:
            acc = jnp.maximum(acc, 0.0)
        if final:
            outs[0][...] = acc
        else:
            outs[0][...] = acc[:, :_H]
            outs[1][...] = acc[:, _H:]

    half = pl.BlockSpec((_BN, _H), lambda i: (i, 0))
    in_specs = [
        half, half,
        half, half,
        half, half,
        pl.BlockSpec((_H, _D), lambda i: (0, 0)),
        pl.BlockSpec((_H, _D), lambda i: (0, 0)),
        pl.BlockSpec((_H, _D), lambda i: (0, 0)),
        pl.BlockSpec((_H, _D), lambda i: (0, 0)),
        pl.BlockSpec((1, _D), lambda i: (0, 0)),
    ]
    if final:
        out_specs = pl.BlockSpec((_BN, _D), lambda i: (i, 0))
        out_shape = jax.ShapeDtypeStruct((_N, _D), jnp.float32)
    else:
        out_specs = [half, half]
        out_shape = [jax.ShapeDtypeStruct((_N, _H), jnp.float32),
                     jax.ShapeDtypeStruct((_N, _H), jnp.float32)]
    return pl.pallas_call(body, grid=(_N // _BN,), in_specs=in_specs,
                          out_specs=out_specs, out_shape=out_shape)


_UPDATE_MID = _make_update(True, False)
_UPDATE_FIN = _make_update(False, True)


def _weights(Wn, Ws, b):
    return (Wn[:, :_H].T, Wn[:, _H:].T, Ws[:, :_H].T, Ws[:, _H:].T,
            b.reshape(1, _D))


def kernel(x, edge_index, W_self_0, W_neigh_0, b_0, W_self_1, W_neigh_1,
           b_1, W_self_2, W_neigh_2, b_2):
    edges = jnp.stack(
        [edge_index[0].reshape(_NSUB, _NCH, _K),
         edge_index[1].reshape(_NSUB, _NCH, _K)], axis=2)
    zrows = jnp.zeros((_K, _H), jnp.float32)
    z40 = jnp.zeros((40, _H), jnp.float32)
    ones40 = jnp.ones((40, _H), jnp.float32)
    dst3 = edge_index[1].reshape(2 * _NSUB, _E // 2 // _NSUB // 40, 40)

    deg_a, deg_b = _DEG(dst3, z40, ones40)
    ha, hb = x[:, :_H], x[:, _H:]
    agg_a, agg_b = _AGG(ha, hb, edges, zrows)
    ha, hb = _UPDATE_MID(agg_a, agg_b, deg_a, deg_b, ha, hb,
                         *_weights(W_neigh_0, W_self_0, b_0))
    agg_a, agg_b = _AGG(ha, hb, edges, zrows)
    ha, hb = _UPDATE_MID(agg_a, agg_b, deg_a, deg_b, ha, hb,
                         *_weights(W_neigh_1, W_self_1, b_1))
    agg_a, agg_b = _AGG(ha, hb, edges, zrows)
    return _UPDATE_FIN(agg_a, agg_b, deg_a, deg_b, ha, hb,
                       *_weights(W_neigh_2, W_self_2, b_2))
